# baseline (speedup 1.0000x reference)
import functools

import jax
import jax.numpy as jnp
from jax import lax
from jax.experimental import pallas as pl
from jax.experimental.pallas import tpu as pltpu
from jax.experimental.pallas import tpu_sc as plsc

B = 4096
H = 50
D = 64
W = 2 * D
VOCAB = 100000
ENV_VOCAB = 1000
NC = 2
NS = 16
NW = NC * NS
BPW = B // NW
NBUF = 5
L = 16
DC = D // L
NU = 16

_MAGIC = 0x5F3759DF


def _inv_norm(s):
    sv = jnp.full((L,), s, jnp.float32)
    i = plsc.bitcast(sv, jnp.int32)
    y = plsc.bitcast(jnp.int32(_MAGIC) - (i >> 1), jnp.float32)
    for _ in range(3):
        y = y * (1.5 - 0.5 * sv * y * y)
    return jnp.minimum(y, 1e12)


def _mesh():
    return plsc.VectorSubcoreMesh(core_axis_name="c", subcore_axis_name="s")


def _make_click_env_kernel():
    @functools.partial(
        pl.kernel,
        out_type=(jax.ShapeDtypeStruct((B, D), jnp.float32),
                  jax.ShapeDtypeStruct((B, D), jnp.float32)),
        mesh=_mesh(),
        compiler_params=pltpu.CompilerParams(needs_layout_passes=False),
        scratch_types=[
            pltpu.VMEM((BPW,), jnp.int32),
            pltpu.VMEM((BPW,), jnp.int32),
            pltpu.VMEM((BPW, W), jnp.float32),
            pltpu.VMEM((BPW, W), jnp.float32),
            pltpu.VMEM((BPW, D), jnp.float32),
            pltpu.VMEM((BPW, D), jnp.float32),
            pltpu.SemaphoreType.DMA,
            pltpu.SemaphoreType.DMA,
        ],
    )
    def k(click_hbm, env_hbm, tclick_hbm, tenv_hbm, click_out, env_out,
          idx_click, idx_env, click_rows, env_rows,
          stage_c, stage_e, sem_c, sem_e):
        wid = lax.axis_index("c") * NS + lax.axis_index("s")
        base = wid * BPW

        pltpu.sync_copy(click_hbm.at[pl.ds(base, BPW)], idx_click)
        pltpu.sync_copy(env_hbm.at[pl.ds(base, BPW)], idx_env)

        click_dma = pltpu.async_copy(tclick_hbm.at[idx_click], click_rows,
                                     sem_c)
        env_dma = pltpu.async_copy(tenv_hbm.at[idx_env], env_rows, sem_e)

        def norm_click(t, c):
            for u in range(NU):
                r = t * NU + u
                vs = [click_rows[r, pl.ds(j * L, L)] for j in range(DC)]
                s16 = vs[0] * vs[0]
                for j in range(1, DC):
                    s16 = s16 + vs[j] * vs[j]
                y = _inv_norm(jnp.sum(s16))
                for j in range(DC):
                    stage_c[r, pl.ds(j * L, L)] = vs[j] * y
            return c

        def envrows(t, c):
            for u in range(NU):
                r = t * NU + u
                for j in range(DC):
                    stage_e[r, pl.ds(j * L, L)] = env_rows[r,
                                                           pl.ds(j * L, L)]
            return c

        click_dma.wait()
        lax.fori_loop(0, BPW // NU, norm_click, 0)
        env_dma.wait()
        lax.fori_loop(0, BPW // NU, envrows, 0)

        pltpu.sync_copy(stage_c, click_out.at[pl.ds(base, BPW)])
        pltpu.sync_copy(stage_e, env_out.at[pl.ds(base, BPW)])

    return k


def _make_hist_kernel():
    @functools.partial(
        pl.kernel,
        out_type=jax.ShapeDtypeStruct((B, D), jnp.float32),
        mesh=_mesh(),
        compiler_params=pltpu.CompilerParams(needs_layout_passes=False,
                                             use_tc_tiling_on_sc=False),
        scratch_types=[
            pltpu.VMEM((BPW, H), jnp.int32),
            pltpu.VMEM((H, BPW), jnp.int32),
            pltpu.VMEM((BPW, D), jnp.float32),
            pltpu.VMEM((BPW, D), jnp.float32),
        ]
        + [pltpu.VMEM((BPW, D), jnp.float32) for _ in range(NBUF)]
        + [pltpu.SemaphoreType.DMA for _ in range(NBUF)],
    )
    def k(hist_hbm, tcount_hbm, out_hbm, idx_raw, idx_hist, acc, stage,
          *bufs_and_sems):
        bufs = bufs_and_sems[:NBUF]
        sems = bufs_and_sems[NBUF:]
        wid = lax.axis_index("c") * NS + lax.axis_index("s")
        base = wid * BPW

        pltpu.sync_copy(hist_hbm.at[pl.ds(base, BPW)], idx_raw)

        lanes = lax.iota(jnp.int32, L)

        def trow(h, c):
            hsplat = jnp.full((L,), h, jnp.int32)
            for g in range(BPW // L):
                v = plsc.load_gather(idx_raw, [g * L + lanes, hsplat])
                idx_hist[h, pl.ds(g * L, L)] = v
            return c

        lax.fori_loop(0, H, trow, 0)

        for b in range(NBUF):
            pltpu.async_copy(tcount_hbm.at[idx_hist.at[b]], bufs[b], sems[b])

        def wait_hist(b):
            pltpu.make_async_copy(tcount_hbm.at[idx_hist.at[0]], bufs[b],
                                  sems[b]).wait()

        def copyrows(buf):
            def body(t, c):
                r0 = t * NU
                for u in range(NU):
                    for j in range(DC):
                        acc[r0 + u, pl.ds(j * L, L)] = buf[r0 + u,
                                                           pl.ds(j * L, L)]
                return c
            return body

        def addrows(buf):
            def body(t, c):
                r0 = t * NU
                for u in range(NU):
                    for j in range(DC):
                        plsc.addupdate(acc.at[r0 + u, pl.ds(j * L, L)],
                                       buf[r0 + u, pl.ds(j * L, L)])
                return c
            return body

        wait_hist(0)
        lax.fori_loop(0, BPW // NU, copyrows(bufs[0]), 0)
        pltpu.async_copy(tcount_hbm.at[idx_hist.at[NBUF]], bufs[0], sems[0])
        for b in range(1, NBUF):
            wait_hist(b)
            lax.fori_loop(0, BPW // NU, addrows(bufs[b]), 0)
            pltpu.async_copy(tcount_hbm.at[idx_hist.at[b + NBUF]], bufs[b],
                             sems[b])

        def outer(g, c):
            for b in range(NBUF):
                h = g * NBUF + b
                wait_hist(b)
                lax.fori_loop(0, BPW // NU, addrows(bufs[b]), 0)

                @pl.when(h + NBUF < H)
                def _():
                    pltpu.async_copy(tcount_hbm.at[idx_hist.at[h + NBUF]],
                                     bufs[b], sems[b])
            return c

        lax.fori_loop(1, H // NBUF, outer, 0)

        def norm_acc(t, c):
            for u in range(NU):
                r = t * NU + u
                vs = [acc[r, pl.ds(j * L, L)] for j in range(DC)]
                s16 = vs[0] * vs[0]
                for j in range(1, DC):
                    s16 = s16 + vs[j] * vs[j]
                y = _inv_norm(jnp.sum(s16))
                for j in range(DC):
                    stage[r, pl.ds(j * L, L)] = vs[j] * y
            return c

        lax.fori_loop(0, BPW // NU, norm_acc, 0)
        pltpu.sync_copy(stage, out_hbm.at[pl.ds(base, BPW)])

    return k


_click_env_kernel = _make_click_env_kernel()
_hist_kernel = _make_hist_kernel()


def kernel(click_article_id, hist_article_ids, user_env,
           table_count, table_click, table_env):
    ci = click_article_id.astype(jnp.int32)
    ui = user_env.astype(jnp.int32)
    hi = hist_article_ids.astype(jnp.int32)
    tcl = jnp.pad(table_click, ((0, 0), (0, D)))
    ten = jnp.pad(table_env, ((0, 0), (0, D)))
    click_vec, env_vec = _click_env_kernel(ci, ui, tcl, ten)
    count_vec = _hist_kernel(hi, table_count)
    return jnp.concatenate([click_vec, count_vec, env_vec], axis=-1)


# speedup vs baseline: 1.0430x; 1.0430x over previous
import functools

import jax
import jax.numpy as jnp
from jax import lax
from jax.experimental import pallas as pl
from jax.experimental.pallas import tpu as pltpu
from jax.experimental.pallas import tpu_sc as plsc

B = 4096
H = 50
D = 64
W = 2 * D
VOCAB = 100000
ENV_VOCAB = 1000
NC = 2
NS = 16
NW = NC * NS
BPW = B // NW
NBUF = 5
L = 16
DC = D // L
NU = 16

_MAGIC = 0x5F3759DF


def _inv_norm(s):
    sv = jnp.full((L,), s, jnp.float32)
    i = plsc.bitcast(sv, jnp.int32)
    y = plsc.bitcast(jnp.int32(_MAGIC) - (i >> 1), jnp.float32)
    for _ in range(3):
        y = y * (1.5 - 0.5 * sv * y * y)
    return jnp.minimum(y, 1e12)


def _mesh():
    return plsc.VectorSubcoreMesh(core_axis_name="c", subcore_axis_name="s")


def _make_click_env_kernel():
    @functools.partial(
        pl.kernel,
        out_type=(jax.ShapeDtypeStruct((B, D), jnp.float32),
                  jax.ShapeDtypeStruct((B, D), jnp.float32)),
        mesh=_mesh(),
        compiler_params=pltpu.CompilerParams(needs_layout_passes=False),
        scratch_types=[
            pltpu.VMEM((BPW,), jnp.int32),
            pltpu.VMEM((BPW,), jnp.int32),
            pltpu.VMEM((BPW,), jnp.int32),
            pltpu.VMEM((BPW,), jnp.int32),
            pltpu.VMEM((BPW, W), jnp.float32),
            pltpu.VMEM((BPW, W), jnp.float32),
            pltpu.VMEM((BPW, D), jnp.float32),
            pltpu.VMEM((BPW, D), jnp.float32),
            pltpu.SemaphoreType.DMA,
            pltpu.SemaphoreType.DMA,
        ],
    )
    def k(click_hbm, env_hbm, tclick_hbm, tenv_hbm, click_out, env_out,
          idx_click, par_click, idx_env, par_env, click_rows, env_rows,
          stage_c, stage_e, sem_c, sem_e):
        wid = lax.axis_index("c") * NS + lax.axis_index("s")
        base = wid * BPW

        pltpu.sync_copy(click_hbm.at[pl.ds(base, BPW)], idx_click)
        pltpu.sync_copy(env_hbm.at[pl.ds(base, BPW)], idx_env)

        for g in range(BPW // L):
            v = idx_click[pl.ds(g * L, L)]
            idx_click[pl.ds(g * L, L)] = v >> 1
            par_click[pl.ds(g * L, L)] = (v & 1) << 6
            u = idx_env[pl.ds(g * L, L)]
            idx_env[pl.ds(g * L, L)] = u >> 1
            par_env[pl.ds(g * L, L)] = (u & 1) << 6

        click_dma = pltpu.async_copy(tclick_hbm.at[idx_click], click_rows,
                                     sem_c)
        env_dma = pltpu.async_copy(tenv_hbm.at[idx_env], env_rows, sem_e)

        def norm_click(t, c):
            pv = par_click[pl.ds(t * NU, L)]
            for u in range(NU):
                r = t * NU + u
                s = pv[u]
                vs = [click_rows[r, pl.ds(s + j * L, L)] for j in range(DC)]
                s16 = vs[0] * vs[0]
                for j in range(1, DC):
                    s16 = s16 + vs[j] * vs[j]
                y = _inv_norm(jnp.sum(s16))
                for j in range(DC):
                    stage_c[r, pl.ds(j * L, L)] = vs[j] * y
            return c

        def envrows(t, c):
            pv = par_env[pl.ds(t * NU, L)]
            for u in range(NU):
                r = t * NU + u
                s = pv[u]
                for j in range(DC):
                    stage_e[r, pl.ds(j * L, L)] = env_rows[r,
                                                           pl.ds(s + j * L, L)]
            return c

        click_dma.wait()
        lax.fori_loop(0, BPW // NU, norm_click, 0)
        env_dma.wait()
        lax.fori_loop(0, BPW // NU, envrows, 0)

        pltpu.sync_copy(stage_c, click_out.at[pl.ds(base, BPW)])
        pltpu.sync_copy(stage_e, env_out.at[pl.ds(base, BPW)])

    return k


def _make_hist_kernel():
    @functools.partial(
        pl.kernel,
        out_type=jax.ShapeDtypeStruct((B, D), jnp.float32),
        mesh=_mesh(),
        compiler_params=pltpu.CompilerParams(needs_layout_passes=False,
                                             use_tc_tiling_on_sc=False),
        scratch_types=[
            pltpu.VMEM((BPW, H), jnp.int32),
            pltpu.VMEM((H, BPW), jnp.int32),
            pltpu.VMEM((BPW, D), jnp.float32),
            pltpu.VMEM((BPW, D), jnp.float32),
        ]
        + [pltpu.VMEM((BPW, D), jnp.float32) for _ in range(NBUF)]
        + [pltpu.SemaphoreType.DMA for _ in range(NBUF)],
    )
    def k(hist_hbm, tcount_hbm, out_hbm, idx_raw, idx_hist, acc, stage,
          *bufs_and_sems):
        bufs = bufs_and_sems[:NBUF]
        sems = bufs_and_sems[NBUF:]
        wid = lax.axis_index("c") * NS + lax.axis_index("s")
        base = wid * BPW

        pltpu.sync_copy(hist_hbm.at[pl.ds(base, BPW)], idx_raw)

        lanes = lax.iota(jnp.int32, L)

        def trow(h, c):
            hsplat = jnp.full((L,), h, jnp.int32)
            for g in range(BPW // L):
                v = plsc.load_gather(idx_raw, [g * L + lanes, hsplat])
                idx_hist[h, pl.ds(g * L, L)] = v
            return c

        lax.fori_loop(0, H, trow, 0)

        for b in range(NBUF):
            pltpu.async_copy(tcount_hbm.at[idx_hist.at[b]], bufs[b], sems[b])

        def wait_hist(b):
            pltpu.make_async_copy(tcount_hbm.at[idx_hist.at[0]], bufs[b],
                                  sems[b]).wait()

        def copyrows(buf):
            def body(t, c):
                r0 = t * NU
                for u in range(NU):
                    for j in range(DC):
                        acc[r0 + u, pl.ds(j * L, L)] = buf[r0 + u,
                                                           pl.ds(j * L, L)]
                return c
            return body

        def addrows(buf):
            def body(t, c):
                r0 = t * NU
                for u in range(NU):
                    for j in range(DC):
                        plsc.addupdate(acc.at[r0 + u, pl.ds(j * L, L)],
                                       buf[r0 + u, pl.ds(j * L, L)])
                return c
            return body

        wait_hist(0)
        lax.fori_loop(0, BPW // NU, copyrows(bufs[0]), 0)
        pltpu.async_copy(tcount_hbm.at[idx_hist.at[NBUF]], bufs[0], sems[0])
        for b in range(1, NBUF):
            wait_hist(b)
            lax.fori_loop(0, BPW // NU, addrows(bufs[b]), 0)
            pltpu.async_copy(tcount_hbm.at[idx_hist.at[b + NBUF]], bufs[b],
                             sems[b])

        def outer(g, c):
            for b in range(NBUF):
                h = g * NBUF + b
                wait_hist(b)
                lax.fori_loop(0, BPW // NU, addrows(bufs[b]), 0)

                @pl.when(h + NBUF < H)
                def _():
                    pltpu.async_copy(tcount_hbm.at[idx_hist.at[h + NBUF]],
                                     bufs[b], sems[b])
            return c

        lax.fori_loop(1, H // NBUF, outer, 0)

        def norm_acc(t, c):
            for u in range(NU):
                r = t * NU + u
                vs = [acc[r, pl.ds(j * L, L)] for j in range(DC)]
                s16 = vs[0] * vs[0]
                for j in range(1, DC):
                    s16 = s16 + vs[j] * vs[j]
                y = _inv_norm(jnp.sum(s16))
                for j in range(DC):
                    stage[r, pl.ds(j * L, L)] = vs[j] * y
            return c

        lax.fori_loop(0, BPW // NU, norm_acc, 0)
        pltpu.sync_copy(stage, out_hbm.at[pl.ds(base, BPW)])

    return k


_click_env_kernel = _make_click_env_kernel()
_hist_kernel = _make_hist_kernel()


def kernel(click_article_id, hist_article_ids, user_env,
           table_count, table_click, table_env):
    ci = click_article_id.astype(jnp.int32)
    ui = user_env.astype(jnp.int32)
    hi = hist_article_ids.astype(jnp.int32)
    tcl = table_click.reshape(VOCAB // 2, W)
    ten = table_env.reshape(ENV_VOCAB // 2, W)
    click_vec, env_vec = _click_env_kernel(ci, ui, tcl, ten)
    count_vec = _hist_kernel(hi, table_count)
    return jnp.concatenate([click_vec, count_vec, env_vec], axis=-1)
